# baseline (speedup 1.0000x reference)
import functools

import numpy as np
import jax
import jax.numpy as jnp
from jax import lax
from jax.experimental import pallas as pl
from jax.experimental.pallas import tpu as pltpu
from jax.experimental.pallas import tpu_sc as plsc

_B = 4096
_NUMD = 13
_F = 26
_VOCAB = 100000
_D = 64


_NC = 2
_NS = 16
_NW = _NC * _NS
_TOT = _B * _F
_PER_W = _TOT // _NW
_RPT = 8
_NT = _F * _VOCAB // _RPT
_BLK = 32
_NBLK = _PER_W // _BLK

_sc_mesh = plsc.VectorSubcoreMesh(core_axis_name="c", subcore_axis_name="s")

_NCH = _PER_W // 128


@functools.partial(
    pl.kernel,
    mesh=_sc_mesh,
    out_type=jax.ShapeDtypeStruct((_TOT, _D), jnp.float32),
    scratch_types=[
        pltpu.VMEM((128,), jnp.int32),
        pltpu.SemaphoreType.DMA,
    ],
    compiler_params=pltpu.CompilerParams(use_tc_tiling_on_sc=True,
                                         needs_layout_passes=False),
)
def _sc_gather(idx_hbm, table_hbm, out_hbm, idx_v, sem):
    wid = lax.axis_index("s") * _NC + lax.axis_index("c")
    base = wid * _PER_W
    c0 = wid * _NCH
    iota16 = lax.iota(jnp.int32, 16)

    def _chunk(p, carry):
        pltpu.sync_copy(idx_hbm.at[wid, p], idx_v)
        f = (c0 + p) // (_B // 128)
        off = f * _VOCAB
        obase = base + p * 128
        for b in range(4):
            cps = []
            for g in range(2):
                o = (b * 2 + g) * 16
                iv = idx_v[pl.ds(o, 16)] + off
                for j in range(16):
                    row = jnp.sum(jnp.where(iota16 == j, iv, 0))
                    cps.append(pltpu.async_copy(
                        table_hbm.at[pl.ds(row, 1)],
                        out_hbm.at[pl.ds(obase + o + j, 1)], sem))
            for cp in cps:
                cp.wait()
        return carry

    lax.fori_loop(0, _NCH, _chunk, 0)


_BS = 256
_NI = _F + 1
_INTER = _NI * (_NI - 1) // 2

_IMAP = np.full((_NI * _NI,), _INTER, np.int32)
for _i in range(1, _NI):
    for _j in range(_i):
        _IMAP[_i * _NI + _j] = _i * (_i - 1) // 2 + _j


def _tc_body(xv_ref, emb_ref, bw0, bb0, bw1, bb1, bw2, bb2,
             w0z, w0d, tb0, tw1, tb1, tw2, tb2, tw3, tb3, out_ref):
    f32 = jnp.float32
    bf16 = jnp.bfloat16
    xv = xv_ref[...].astype(bf16)
    h = jnp.maximum(jnp.dot(xv, bw0[...], preferred_element_type=f32) + bb0[...], 0.0)
    h = jnp.maximum(jnp.dot(h.astype(bf16), bw1[...], preferred_element_type=f32) + bb1[...], 0.0)
    dense = jnp.maximum(jnp.dot(h.astype(bf16), bw2[...], preferred_element_type=f32) + bb2[...], 0.0)

    emb = emb_ref[...]
    t = jnp.concatenate([dense[None, :, :], emb], axis=0).astype(bf16)
    z = lax.dot_general(t, t, (((2,), (2,)), ((1,), (1,))),
                        preferred_element_type=f32)
    zf = z.reshape(_BS, _NI * _NI).astype(bf16)
    h = jnp.dot(zf, w0z[...], preferred_element_type=f32)
    h += jnp.dot(dense.astype(bf16), w0d[...], preferred_element_type=f32)
    h = jnp.maximum(h + tb0[...], 0.0)
    h = jnp.maximum(jnp.dot(h.astype(bf16), tw1[...], preferred_element_type=f32) + tb1[...], 0.0)
    h = jnp.maximum(jnp.dot(h.astype(bf16), tw2[...], preferred_element_type=f32) + tb2[...], 0.0)
    logit = jnp.dot(h.astype(bf16), tw3[...], preferred_element_type=f32) + tb3[...]
    out_ref[...] = jax.nn.sigmoid(logit)


def _full2d(shape):
    return pl.BlockSpec(shape, lambda i: (0, 0))


def kernel(Xi, Xv, emb_tables, bW0, bb0, bW1, bb1, bW2, bb2,
           tW0, tb0, tW1, tb1, tW2, tb2, tW3, tb3):
    xi32 = Xi.astype(jnp.int32)
    idx3d = xi32.T.reshape(_NW, _PER_W // 128, 128)
    table_rows = emb_tables.reshape(_F * _VOCAB, _D)
    emb_flat = _sc_gather(idx3d, table_rows)
    embT = emb_flat.reshape(_F, _B, _D)

    bf16 = jnp.bfloat16
    w0ext = jnp.concatenate(
        [tW0[:_INTER], jnp.zeros((1, tW0.shape[1]), tW0.dtype)], axis=0)
    w0z = w0ext[_IMAP].astype(bf16)
    w0d = tW0[_INTER:].astype(bf16)

    grid = (_B // _BS,)
    weights = [bW0.astype(bf16), bb0.reshape(1, -1),
               bW1.astype(bf16), bb1.reshape(1, -1),
               bW2.astype(bf16), bb2.reshape(1, -1),
               w0z, w0d, tb0.reshape(1, -1),
               tW1.astype(bf16), tb1.reshape(1, -1),
               tW2.astype(bf16), tb2.reshape(1, -1),
               tW3.astype(bf16), tb3.reshape(1, -1)]
    w_specs = [_full2d(w.shape) for w in weights]
    out = pl.pallas_call(
        _tc_body,
        grid=grid,
        in_specs=[
            pl.BlockSpec((_BS, _NUMD), lambda i: (i, 0)),
            pl.BlockSpec((_F, _BS, _D), lambda i: (0, i, 0)),
            *w_specs,
        ],
        out_specs=pl.BlockSpec((_BS, 1), lambda i: (i, 0)),
        out_shape=jax.ShapeDtypeStruct((_B, 1), jnp.float32),
    )(Xv, embT, *weights)
    return out


# speedup vs baseline: 1.3482x; 1.3482x over previous
import functools

import numpy as np
import jax
import jax.numpy as jnp
from jax import lax
from jax.experimental import pallas as pl
from jax.experimental.pallas import tpu as pltpu
from jax.experimental.pallas import tpu_sc as plsc

_B = 4096
_NUMD = 13
_F = 26
_VOCAB = 100000
_D = 64


_NC = 2
_NS = 16
_NW = _NC * _NS
_TOT = _B * _F
_PER_W = _TOT // _NW
_RPT = 8
_NT = _F * _VOCAB // _RPT
_BLK = 32
_NBLK = _PER_W // _BLK

_sc_mesh = plsc.VectorSubcoreMesh(core_axis_name="c", subcore_axis_name="s")

_NCH = _PER_W // 128
_PAIRS = _F * _VOCAB // 2


@functools.partial(
    pl.kernel,
    mesh=_sc_mesh,
    out_type=jax.ShapeDtypeStruct((_TOT, 2 * _D), jnp.float32),
    scratch_types=[
        pltpu.VMEM((_NCH, 128), jnp.int32),
        pltpu.VMEM((128, 2 * _D), jnp.float32),
        pltpu.VMEM((128, 2 * _D), jnp.float32),
        pltpu.SemaphoreType.DMA,
        pltpu.SemaphoreType.DMA,
    ],
    compiler_params=pltpu.CompilerParams(use_tc_tiling_on_sc=True),
)
def _sc_gather(idx_hbm, table_hbm, out_hbm, idx_v, buf0, buf1, sem0, sem1):
    wid = lax.axis_index("s") * _NC + lax.axis_index("c")
    base = wid * _PER_W
    c0 = wid * _NCH
    pltpu.sync_copy(idx_hbm.at[wid], idx_v)

    def _chunk(r, buf, sem):
        f = (c0 + r) // (_B // 128)
        off = f * _VOCAB
        cps = []
        for g in range(8):
            vec = idx_v[r, pl.ds(g * 16, 16)] + off
            pvec = lax.shift_right_logical(vec, 1)
            cps.append(pltpu.async_copy(
                table_hbm.at[pvec], buf.at[pl.ds(g * 16, 16)], sem))
        return cps

    def _pair_body(p, carry):
        r0 = p * 2
        cps0 = _chunk(r0, buf0, sem0)
        cps1 = _chunk(r0 + 1, buf1, sem1)
        for cp in cps0:
            cp.wait()
        pltpu.sync_copy(buf0, out_hbm.at[pl.ds(base + r0 * 128, 128)])
        for cp in cps1:
            cp.wait()
        pltpu.sync_copy(buf1, out_hbm.at[pl.ds(base + (r0 + 1) * 128, 128)])
        return carry

    lax.fori_loop(0, _NCH // 2, _pair_body, 0)


_BS = 256
_NI = _F + 1
_INTER = _NI * (_NI - 1) // 2

_IMAP = np.full((_NI * _NI,), _INTER, np.int32)
for _i in range(1, _NI):
    for _j in range(_i):
        _IMAP[_i * _NI + _j] = _i * (_i - 1) // 2 + _j


def _tc_body(xv_ref, emb_ref, par_ref, bw0, bb0, bw1, bb1, bw2, bb2,
             w0z, w0d, tb0, tw1, tb1, tw2, tb2, tw3, tb3, out_ref):
    f32 = jnp.float32
    bf16 = jnp.bfloat16
    xv = xv_ref[...].astype(bf16)
    h = jnp.maximum(jnp.dot(xv, bw0[...], preferred_element_type=f32) + bb0[...], 0.0)
    h = jnp.maximum(jnp.dot(h.astype(bf16), bw1[...], preferred_element_type=f32) + bb1[...], 0.0)
    dense = jnp.maximum(jnp.dot(h.astype(bf16), bw2[...], preferred_element_type=f32) + bb2[...], 0.0)

    emb2 = emb_ref[...]
    odd = (par_ref[...][:, :, None] & 1) == 1
    emb = jnp.where(odd, emb2[:, :, _D:], emb2[:, :, :_D])
    t = jnp.concatenate([dense[None, :, :], emb], axis=0).astype(bf16)
    z = lax.dot_general(t, t, (((2,), (2,)), ((1,), (1,))),
                        preferred_element_type=f32)
    zf = z.reshape(_BS, _NI * _NI).astype(bf16)
    h = jnp.dot(zf, w0z[...], preferred_element_type=f32)
    h += jnp.dot(dense.astype(bf16), w0d[...], preferred_element_type=f32)
    h = jnp.maximum(h + tb0[...], 0.0)
    h = jnp.maximum(jnp.dot(h.astype(bf16), tw1[...], preferred_element_type=f32) + tb1[...], 0.0)
    h = jnp.maximum(jnp.dot(h.astype(bf16), tw2[...], preferred_element_type=f32) + tb2[...], 0.0)
    logit = jnp.dot(h.astype(bf16), tw3[...], preferred_element_type=f32) + tb3[...]
    out_ref[...] = jax.nn.sigmoid(logit)


def _full2d(shape):
    return pl.BlockSpec(shape, lambda i: (0, 0))


def kernel(Xi, Xv, emb_tables, bW0, bb0, bW1, bb1, bW2, bb2,
           tW0, tb0, tW1, tb1, tW2, tb2, tW3, tb3):
    xi32 = Xi.astype(jnp.int32)
    xiT = xi32.T
    idx3d = xiT.reshape(_NW, _PER_W // 128, 128)
    table_pairs = emb_tables.reshape(_PAIRS, 2 * _D)
    emb_flat = _sc_gather(idx3d, table_pairs)
    embT = emb_flat.reshape(_F, _B, 2 * _D)

    bf16 = jnp.bfloat16
    w0ext = jnp.concatenate(
        [tW0[:_INTER], jnp.zeros((1, tW0.shape[1]), tW0.dtype)], axis=0)
    w0z = w0ext[_IMAP].astype(bf16)
    w0d = tW0[_INTER:].astype(bf16)

    grid = (_B // _BS,)
    weights = [bW0.astype(bf16), bb0.reshape(1, -1),
               bW1.astype(bf16), bb1.reshape(1, -1),
               bW2.astype(bf16), bb2.reshape(1, -1),
               w0z, w0d, tb0.reshape(1, -1),
               tW1.astype(bf16), tb1.reshape(1, -1),
               tW2.astype(bf16), tb2.reshape(1, -1),
               tW3.astype(bf16), tb3.reshape(1, -1)]
    w_specs = [_full2d(w.shape) for w in weights]
    out = pl.pallas_call(
        _tc_body,
        grid=grid,
        in_specs=[
            pl.BlockSpec((_BS, _NUMD), lambda i: (i, 0)),
            pl.BlockSpec((_F, _BS, 2 * _D), lambda i: (0, i, 0)),
            pl.BlockSpec((_F, _BS), lambda i: (0, i)),
            *w_specs,
        ],
        out_specs=pl.BlockSpec((_BS, 1), lambda i: (i, 0)),
        out_shape=jax.ShapeDtypeStruct((_B, 1), jnp.float32),
    )(Xv, embT, xiT, *weights)
    return out
